# baseline (speedup 1.0000x reference)
import functools

import jax
import jax.numpy as jnp
from jax import lax
from jax.experimental import pallas as pl
from jax.experimental.pallas import tpu as pltpu
from jax.experimental.pallas import tpu_sc as plsc

B, T, L, H, D = 64, 64, 577, 12, 768
LNC = L - 1
K = LNC // 2
KP1 = K + 1
NIDX = 512
CHUNK = 128
NW = 32


def _score_kernel(ca_ref, tm_ref, im_ref, ts_ref, idx_ref, mask_ref):
    X = ca_ref[0]

    acc = X[:, 0:8, :]
    for g in range(1, 8):
        acc = acc + X[:, 8 * g:8 * g + 8, :]
    t1 = acc[:, 4:8, :] + acc[:, 0:4, :]
    t2 = t1[:, 2:4, :] + t1[:, 0:2, :]
    S = t2[:, 1, :] + t2[:, 0, :]

    t_len = jnp.sum(tm_ref[...])
    Sd = S / t_len

    ts_ref[0] = Sd.T

    A = Sd[0:4, :] + Sd[8:12, :]
    a = Sd[4:8, :] + A
    bb = a[2:4, :] + a[0:2, :]
    c = bb[1:2, :] + bb[0:1, :]
    scores = c / jnp.float32(12.0)
    s_nc = scores[:, 1:]

    col = s_nc.T

    krow = jnp.broadcast_to(
        lax.bitcast_convert_type(s_nc, jnp.int32), (LNC, LNC))
    kcol = jnp.broadcast_to(
        lax.bitcast_convert_type(col, jnp.int32), (LNC, LNC))
    ii = lax.broadcasted_iota(jnp.int32, (LNC, LNC), 0)
    jj = lax.broadcasted_iota(jnp.int32, (LNC, LNC), 1)
    tri = jnp.where(jj < ii, 1, 0)
    G = (krow - kcol + tri) > 0
    ones = jnp.ones((LNC, 1), jnp.int8)
    rank = lax.dot_general(G.astype(jnp.int8), ones,
                           (((1,), (0,)), ((), ())),
                           preferred_element_type=jnp.int32)

    NP = 384
    rr_l = lax.broadcasted_iota(jnp.int32, (LNC, NP), 1)
    rr = jnp.where(rr_l < KP1, rr_l, rr_l - KP1)
    P2 = (jnp.broadcast_to(rank + 1, (LNC, NP)) == rr).astype(jnp.int8)

    idx_i = lax.broadcasted_iota(jnp.int32, (1, LNC), 1)
    idx_hi = (idx_i // 64).astype(jnp.int8)
    idx_lo = (idx_i % 64).astype(jnp.int8)
    mask_row = im_ref[0][:, 1:].astype(jnp.int8)
    lhs = jnp.concatenate([idx_hi, idx_lo, mask_row], axis=0)
    packed = lax.dot_general(lhs, P2, (((1,), (0,)), ((), ())),
                             preferred_element_type=jnp.int32)
    top_shift = packed[0:1, :] * 64 + packed[1:2, :]
    gmask = packed[2:3, :].astype(jnp.float32)

    lane = lax.broadcasted_iota(jnp.int32, (1, NP), 1)
    lane_t = jnp.where(lane < KP1, lane, lane - KP1)
    gidx = jnp.where(lane_t == 0, 0, 1 + top_shift)
    gidx = jnp.concatenate(
        [gidx, jnp.zeros((1, NIDX - NP), jnp.int32)], axis=1)
    idx_ref[...] = gidx[0]

    cls_mask = im_ref[0, 0, 0]
    mrow = jnp.where(lane == 0, cls_mask, gmask)
    mask_ref[0] = mrow[:, :KP1]


def _tc_scores(cross_attn, text_mask, image_mask):
    return pl.pallas_call(
        _score_kernel,
        grid=(B,),
        in_specs=[
            pl.BlockSpec((1, H, T, L), lambda b: (b, 0, 0, 0)),
            pl.BlockSpec((1, 1, T), lambda b: (b, 0, 0)),
            pl.BlockSpec((1, 1, L), lambda b: (b, 0, 0)),
        ],
        out_specs=[
            pl.BlockSpec((1, L, H), lambda b: (b, 0, 0)),
            pl.BlockSpec((NIDX,), lambda b: (b,)),
            pl.BlockSpec((1, 1, KP1), lambda b: (b, 0, 0)),
        ],
        out_shape=[
            jax.ShapeDtypeStruct((B, L, H), jnp.float32),
            jax.ShapeDtypeStruct((B * NIDX,), jnp.int32),
            jax.ShapeDtypeStruct((B, 1, KP1), jnp.float32),
        ],
    )(cross_attn, text_mask.reshape(B, 1, T), image_mask.reshape(B, 1, L))


GCH = 64
NCH = 5
NTAIL = KP1 - 4 * GCH


def _make_sc_gather():
    mesh = plsc.VectorSubcoreMesh(core_axis_name="c", subcore_axis_name="s")

    @functools.partial(
        pl.kernel,
        mesh=mesh,
        out_type=jax.ShapeDtypeStruct((B, KP1, D), jnp.float32),
        scratch_types=[
            pltpu.VMEM((3 * CHUNK,), jnp.int32),
            pltpu.VMEM((GCH, D), jnp.float32),
            pltpu.VMEM((GCH, D), jnp.float32),
            pltpu.VMEM((GCH,), jnp.int32),
            pltpu.SemaphoreType.DMA,
            pltpu.SemaphoreType.DMA,
        ],
    )
    def sc_gather(img_hbm, idx_hbm, out_hbm, idx_v, rows_a, rows_b, wtail,
                  sem_a, sem_b):
        w = lax.axis_index("s") * 2 + lax.axis_index("c")

        i16 = lax.broadcasted_iota(jnp.int32, (16,), 0)
        for k in range(GCH // 16):
            v = i16 + 16 * k
            wtail[pl.ds(16 * k, 16)] = jnp.where(
                v < NTAIL, v + 4 * GCH, v - NTAIL)

        bufs = (rows_a, rows_b)
        sems = (sem_a, sem_b)
        for bo in range(2):
            b = w * 2 + bo
            imgb = img_hbm.at[b]
            outb = out_hbm.at[b]
            for k in range(3):
                pltpu.sync_copy(
                    idx_hbm.at[pl.ds(b * NIDX + k * CHUNK, CHUNK)],
                    idx_v.at[pl.ds(k * CHUNK, CHUNK)])

            copies = []
            for ch in range(NCH):
                copies.append(pltpu.make_async_copy(
                    imgb.at[idx_v.at[pl.ds(ch * GCH, GCH)]],
                    bufs[ch % 2], sems[ch % 2]))
            copies[0].start()
            for ch in range(NCH):
                if ch + 1 < NCH:
                    copies[ch + 1].start()
                copies[ch].wait()
                if ch < NCH - 1:
                    pltpu.sync_copy(bufs[ch % 2],
                                    outb.at[pl.ds(ch * GCH, GCH)])
                else:
                    pltpu.async_copy(bufs[ch % 2], outb.at[wtail],
                                     sems[ch % 2]).wait()

    return sc_gather


_SC_GATHER_CACHE = []


def kernel(layer_idx, text_states, text_mask, image_states, image_mask,
           cross_attn, previous_keep_mask):
    token_scores, gidx, new_mask = _tc_scores(cross_attn, text_mask, image_mask)
    if not _SC_GATHER_CACHE:
        _SC_GATHER_CACHE.append(_make_sc_gather())
    new_img_states = _SC_GATHER_CACHE[0](image_states, gidx)
    new_img_mask = new_mask.reshape(B, KP1)
    return (new_img_states, new_img_mask, previous_keep_mask, token_scores)
